# baseline (speedup 1.0000x reference)
import functools

import jax
import jax.numpy as jnp
from jax import lax
from jax.experimental import pallas as pl
from jax.experimental.pallas import tpu as pltpu
from jax.experimental.pallas import tpu_sc as plsc

try:
    _INFO = plsc.get_sparse_core_info()
    _NC = _INFO.num_cores
    _NS = _INFO.num_subcores
except Exception:
    _NC, _NS = 2, 16
_NW = _NC * _NS
_LANES = 16

_CR = 192


def _add_chunk(xref, tref, CR):
    def row_body(r, c):
        for g in range(128 // _LANES):
            sl = pl.ds(g * _LANES, _LANES)
            plsc.addupdate(xref.at[r, sl], tref[r, sl])
        return c

    lax.fori_loop(0, CR, row_body, 0, unroll=2)


def _tc_body(x_ref, t_ref, o_ref):
    o_ref[...] = x_ref[...] + t_ref[...][None]


def _tc_add(x, emb_table, b_lo, block_s):
    B, S, D = x.shape
    n_b = B - b_lo
    n_s = S // block_s
    return pl.pallas_call(
        _tc_body,
        grid=(n_s, n_b),
        in_specs=[
            pl.BlockSpec((1, block_s, D), lambda i, j: (j + b_lo, i, 0)),
            pl.BlockSpec((block_s, D), lambda i, j: (i, 0)),
        ],
        out_specs=pl.BlockSpec((1, block_s, D), lambda i, j: (j, i, 0)),
        out_shape=jax.ShapeDtypeStruct((n_b, S, D), x.dtype),
    )(x, emb_table)


def _sc_add(x3, t2, B, M):
    rows_per_w = M // _NW
    n_chunks = rows_per_w // _CR
    n_pairs = n_chunks // 2

    @functools.partial(
        pl.kernel,
        mesh=plsc.VectorSubcoreMesh(core_axis_name="c", subcore_axis_name="s"),
        out_type=jax.ShapeDtypeStruct((B, M, 128), jnp.float32),
        scratch_types=[
            pltpu.VMEM((_CR, 128), jnp.float32),
            pltpu.VMEM((_CR, 128), jnp.float32),
            pltpu.VMEM((_CR, 128), jnp.float32),
            pltpu.VMEM((_CR, 128), jnp.float32),
            pltpu.SemaphoreType.DMA,
            pltpu.SemaphoreType.DMA,
            pltpu.SemaphoreType.DMA,
            pltpu.SemaphoreType.DMA,
            pltpu.SemaphoreType.DMA,
            pltpu.SemaphoreType.DMA,
        ],
    )
    def run(x_hbm, t_hbm, o_hbm, tbuf0, tbuf1, xbuf0, xbuf1,
            tsem0, tsem1, xsem0, xsem1, osem0, osem1):
        wid = lax.axis_index("s") * _NC + lax.axis_index("c")
        base = wid * rows_per_w
        tb, tsem = (tbuf0, tbuf1), (tsem0, tsem1)
        xb, xsem = (xbuf0, xbuf1), (xsem0, xsem1)
        osem = (osem0, osem1)

        def row0(k):
            return pl.multiple_of(base + k * _CR, 8)

        def t_load(k, kp):
            return pltpu.make_async_copy(
                t_hbm.at[pl.ds(row0(k), _CR), :], tb[kp], tsem[kp])

        def x_load(k, b, p):
            return pltpu.make_async_copy(
                x_hbm.at[b, pl.ds(row0(k), _CR), :], xb[p], xsem[p])

        def o_store(k, b, p):
            return pltpu.make_async_copy(
                xb[p], o_hbm.at[b, pl.ds(row0(k), _CR), :], osem[p])

        t_load(0, 0).start()
        x_load(0, 0, 0).start()

        def pair_body(kk, carry):
            for kp in range(2):
                k = kk * 2 + kp
                for b in range(B):
                    p = b % 2
                    q = (b + 1) % 2
                    if b == 0:
                        if kp == 0:
                            t_load(k + 1, 1).start()
                        else:
                            @pl.when(kk < n_pairs - 1)
                            def _():
                                t_load(k + 1, 0).start()
                        t_load(k, kp).wait()
                    if b == 0:
                        if kp == 1:
                            o_store(k - 1, B - 1, q).wait()
                        else:
                            @pl.when(kk > 0)
                            def _():
                                o_store(k - 1, B - 1, q).wait()
                        x_load(k, 1, q).start()
                    elif b < B - 1:
                        o_store(k, b - 1, q).wait()
                        x_load(k, b + 1, q).start()
                    else:
                        if kp == 0:
                            o_store(k, b - 1, q).wait()
                            x_load(k + 1, 0, q).start()
                        else:
                            @pl.when(kk < n_pairs - 1)
                            def _():
                                o_store(k, b - 1, q).wait()
                                x_load(k + 1, 0, q).start()
                    x_load(k, b, p).wait()
                    _add_chunk(xb[p], tb[kp], _CR)
                    o_store(k, b, p).start()
            return carry

        lax.fori_loop(0, n_pairs, pair_body, 0)

        o_store(n_chunks - 1, B - 2, (B - 2) % 2).wait()
        o_store(n_chunks - 1, B - 1, (B - 1) % 2).wait()

    return run(x3, t2)


def _to_linear_view(a):
    s, d = a.shape[-2], a.shape[-1]
    lead = a.shape[:-2]
    a5 = a.reshape(*lead, s // 8, 8, d // 128, 128)
    perm = tuple(range(len(lead))) + tuple(
        len(lead) + i for i in (0, 2, 1, 3))
    return a5.transpose(perm).reshape(*lead, s * d // 128, 128)


def _from_linear_view(a3, s, d):
    lead = a3.shape[:-2]
    a5 = a3.reshape(*lead, s // 8, d // 128, 8, 128)
    perm = tuple(range(len(lead))) + tuple(
        len(lead) + i for i in (0, 2, 1, 3))
    return a5.transpose(perm).reshape(*lead, s, d)


_NB_SC = 2


def kernel(x, emb_table):
    B, S, D = x.shape
    M = S * D // 128
    x3 = _to_linear_view(x)
    t2 = _to_linear_view(emb_table)
    sc3 = _sc_add(x3, t2, _NB_SC, M)
    sc_out = _from_linear_view(sc3, S, D)
    tc_out = _tc_add(x, emb_table, _NB_SC, 1024)
    return jnp.concatenate([sc_out, tc_out], axis=0)


# speedup vs baseline: 1.6564x; 1.6564x over previous
import functools

import jax
import jax.numpy as jnp
from jax import lax
from jax.experimental import pallas as pl
from jax.experimental.pallas import tpu as pltpu
from jax.experimental.pallas import tpu_sc as plsc

try:
    _INFO = plsc.get_sparse_core_info()
    _NC = _INFO.num_cores
    _NS = _INFO.num_subcores
except Exception:
    _NC, _NS = 2, 16
_NW = _NC * _NS
_LANES = 16

_CR = 128


def _add_chunk(xref, tref, CR):
    def row_body(r, c):
        for g in range(128 // _LANES):
            sl = pl.ds(g * _LANES, _LANES)
            plsc.addupdate(xref.at[r, sl], tref[r, sl])
        return c

    lax.fori_loop(0, CR, row_body, 0, unroll=2)


def _sc_add(x3, t2, B, M):
    rows_per_w = M // _NW
    n_chunks = rows_per_w // _CR
    n_pairs = n_chunks // 2

    @functools.partial(
        pl.kernel,
        mesh=plsc.VectorSubcoreMesh(core_axis_name="c", subcore_axis_name="s"),
        out_type=jax.ShapeDtypeStruct((B, M, 128), jnp.float32),
        scratch_types=[
            pltpu.VMEM((_CR, 128), jnp.float32),
            pltpu.VMEM((_CR, 128), jnp.float32),
            pltpu.VMEM((_CR, 128), jnp.float32),
            pltpu.VMEM((_CR, 128), jnp.float32),
            pltpu.VMEM((_CR, 128), jnp.float32),
            pltpu.VMEM((_CR, 128), jnp.float32),
            pltpu.SemaphoreType.DMA,
            pltpu.SemaphoreType.DMA,
            pltpu.SemaphoreType.DMA,
            pltpu.SemaphoreType.DMA,
            pltpu.SemaphoreType.DMA,
            pltpu.SemaphoreType.DMA,
            pltpu.SemaphoreType.DMA,
            pltpu.SemaphoreType.DMA,
        ],
    )
    def run(x_hbm, t_hbm, o_hbm, tbuf0, tbuf1, xbuf0, xbuf1, xbuf2, xbuf3,
            tsem0, tsem1, xsem0, xsem1, xsem2, xsem3, osem0, osem1):
        wid = lax.axis_index("s") * _NC + lax.axis_index("c")
        base = wid * rows_per_w
        tb, tsem = (tbuf0, tbuf1), (tsem0, tsem1)
        xb = (xbuf0, xbuf1, xbuf2, xbuf3)
        xsem = (xsem0, xsem1, xsem2, xsem3)
        osem = (osem0, osem1)

        def row0(k):
            return pl.multiple_of(base + k * _CR, 8)

        def t_load(k, kp):
            return pltpu.make_async_copy(
                t_hbm.at[pl.ds(row0(k), _CR), :], tb[kp], tsem[kp])

        def x_load(k, b):
            return pltpu.make_async_copy(
                x_hbm.at[b, pl.ds(row0(k), _CR), :], xb[b], xsem[b])

        def o_store(k, b):
            return pltpu.make_async_copy(
                xb[b], o_hbm.at[b, pl.ds(row0(k), _CR), :], osem[b % 2])

        t_load(0, 0).start()
        x_load(0, 0).start()
        x_load(0, 1).start()

        def pair_body(kk, carry):
            for kp in range(2):
                k = kk * 2 + kp
                for b in range(B):
                    if b == 0:
                        if kp == 0:
                            t_load(k + 1, 1).start()
                        else:
                            @pl.when(kk < n_pairs - 1)
                            def _():
                                t_load(k + 1, 0).start()
                        t_load(k, kp).wait()
                    nb = (b + 2) % 4
                    nk = k if b < 2 else k + 1
                    if b < 2:
                        if kp == 0:
                            @pl.when(kk > 0)
                            def _():
                                o_store(k - 1, nb).wait()
                        else:
                            o_store(k - 1, nb).wait()
                        x_load(nk, nb).start()
                    else:
                        o_store(k, nb).wait()
                        if kp == 0:
                            x_load(nk, nb).start()
                        else:
                            @pl.when(kk < n_pairs - 1)
                            def _():
                                x_load(nk, nb).start()
                    x_load(k, b).wait()
                    _add_chunk(xb[b], tb[kp], _CR)
                    o_store(k, b).start()
            return carry

        lax.fori_loop(0, n_pairs, pair_body, 0)

        o_store(n_chunks - 1, 2).wait()
        o_store(n_chunks - 1, 3).wait()

    return run(x3, t2)


def _to_linear_view(a):
    s, d = a.shape[-2], a.shape[-1]
    lead = a.shape[:-2]
    a5 = a.reshape(*lead, s // 8, 8, d // 128, 128)
    perm = tuple(range(len(lead))) + tuple(
        len(lead) + i for i in (0, 2, 1, 3))
    return a5.transpose(perm).reshape(*lead, s * d // 128, 128)


def _from_linear_view(a3, s, d):
    lead = a3.shape[:-2]
    a5 = a3.reshape(*lead, s // 8, d // 128, 8, 128)
    perm = tuple(range(len(lead))) + tuple(
        len(lead) + i for i in (0, 2, 1, 3))
    return a5.transpose(perm).reshape(*lead, s, d)


def kernel(x, emb_table):
    B, S, D = x.shape
    M = S * D // 128
    x3 = _to_linear_view(x)
    t2 = _to_linear_view(emb_table)
    out3 = _sc_add(x3, t2, B, M)
    return _from_linear_view(out3, S, D)


# speedup vs baseline: 1.6690x; 1.0076x over previous
import functools

import jax
import jax.numpy as jnp
from jax import lax
from jax.experimental import pallas as pl
from jax.experimental.pallas import tpu as pltpu
from jax.experimental.pallas import tpu_sc as plsc

try:
    _INFO = plsc.get_sparse_core_info()
    _NC = _INFO.num_cores
    _NS = _INFO.num_subcores
except Exception:
    _NC, _NS = 2, 16
_NW = _NC * _NS
_LANES = 16

_CR = 96


def _add_chunk(xref, tref, CR):
    def row_body(r, c):
        for g in range(128 // _LANES):
            sl = pl.ds(g * _LANES, _LANES)
            plsc.addupdate(xref.at[r, sl], tref[r, sl])
        return c

    lax.fori_loop(0, CR, row_body, 0, unroll=2)


def _sc_add(x3, t2, B, M):
    rows_per_w = M // _NW
    n_chunks = rows_per_w // _CR
    n_pairs = n_chunks // 2

    @functools.partial(
        pl.kernel,
        mesh=plsc.VectorSubcoreMesh(core_axis_name="c", subcore_axis_name="s"),
        out_type=jax.ShapeDtypeStruct((B, M, 128), jnp.float32),
        scratch_types=(
            [pltpu.VMEM((_CR, 128), jnp.float32)] * 10
            + [pltpu.SemaphoreType.DMA] * 8
        ),
    )
    def run(x_hbm, t_hbm, o_hbm, tbuf0, tbuf1,
            xbuf0, xbuf1, xbuf2, xbuf3, xbuf4, xbuf5, xbuf6, xbuf7,
            tsem0, tsem1, xsem0, xsem1, xsem2, xsem3, osem0, osem1):
        wid = lax.axis_index("s") * _NC + lax.axis_index("c")
        base = wid * rows_per_w
        tb, tsem = (tbuf0, tbuf1), (tsem0, tsem1)
        xb = (xbuf0, xbuf1, xbuf2, xbuf3, xbuf4, xbuf5, xbuf6, xbuf7)
        xsem = (xsem0, xsem1, xsem2, xsem3)
        osem = (osem0, osem1)

        def row0(k):
            return pl.multiple_of(base + k * _CR, 8)

        def t_load(k, kp):
            return pltpu.make_async_copy(
                t_hbm.at[pl.ds(row0(k), _CR), :], tb[kp], tsem[kp])

        def x_load(k, b, kp):
            return pltpu.make_async_copy(
                x_hbm.at[b, pl.ds(row0(k), _CR), :], xb[4 * kp + b], xsem[b])

        def o_store(k, b, kp):
            return pltpu.make_async_copy(
                xb[4 * kp + b], o_hbm.at[b, pl.ds(row0(k), _CR), :],
                osem[b % 2])

        t_load(0, 0).start()
        for b in range(B):
            x_load(0, b, 0).start()

        def pair_body(kk, carry):
            for kp in range(2):
                k = kk * 2 + kp
                kq = 1 - kp
                for b in range(B):
                    if b == 0:
                        if kp == 0:
                            t_load(k + 1, 1).start()
                        else:
                            @pl.when(kk < n_pairs - 1)
                            def _():
                                t_load(k + 1, 0).start()
                        t_load(k, kp).wait()
                    if kp == 0:
                        @pl.when(kk > 0)
                        def _():
                            o_store(k - 1, b, kq).wait()
                        x_load(k + 1, b, kq).start()
                    else:
                        o_store(k - 1, b, kq).wait()

                        @pl.when(kk < n_pairs - 1)
                        def _():
                            x_load(k + 1, b, kq).start()
                    x_load(k, b, kp).wait()
                    _add_chunk(xb[4 * kp + b], tb[kp], _CR)
                    o_store(k, b, kp).start()
            return carry

        lax.fori_loop(0, n_pairs, pair_body, 0)

        for b in range(B):
            o_store(n_chunks - 1, b, 1).wait()

    return run(x3, t2)


def _to_linear_view(a):
    s, d = a.shape[-2], a.shape[-1]
    lead = a.shape[:-2]
    a5 = a.reshape(*lead, s // 8, 8, d // 128, 128)
    perm = tuple(range(len(lead))) + tuple(
        len(lead) + i for i in (0, 2, 1, 3))
    return a5.transpose(perm).reshape(*lead, s * d // 128, 128)


def _from_linear_view(a3, s, d):
    lead = a3.shape[:-2]
    a5 = a3.reshape(*lead, s // 8, d // 128, 8, 128)
    perm = tuple(range(len(lead))) + tuple(
        len(lead) + i for i in (0, 2, 1, 3))
    return a5.transpose(perm).reshape(*lead, s, d)


def kernel(x, emb_table):
    B, S, D = x.shape
    M = S * D // 128
    x3 = _to_linear_view(x)
    t2 = _to_linear_view(emb_table)
    out3 = _sc_add(x3, t2, B, M)
    return _from_linear_view(out3, S, D)
